# baseline (speedup 1.0000x reference)
import functools

import jax
import jax.numpy as jnp
from jax import lax
from jax.experimental import pallas as pl
from jax.experimental.pallas import tpu as pltpu
from jax.experimental.pallas import tpu_sc as plsc

D_MODEL = 64
SCALE = 8.0
G = 128
NW = 32
L = 16


def _build(n_groups):
    mesh = plsc.VectorSubcoreMesh(core_axis_name="c", subcore_axis_name="s")

    @functools.partial(
        pl.kernel,
        mesh=mesh,
        compiler_params=pltpu.CompilerParams(use_tc_tiling_on_sc=False),
        out_type=jax.ShapeDtypeStruct((NW, n_groups, G, D_MODEL), jnp.float32),
        scratch_types=[
            pltpu.VMEM((n_groups, G), jnp.int32),
            pltpu.VMEM((G, D_MODEL), jnp.float32),
            pltpu.SemaphoreType.DMA,
        ],
    )
    def emb_kernel(x_hbm, lut_hbm, out_hbm, idx_v, rows_v, gsem):
        wid = lax.axis_index("s") * 2 + lax.axis_index("c")
        pltpu.sync_copy(x_hbm.at[wid], idx_v)

        def group_body(g, carry):
            pltpu.async_copy(lut_hbm.at[idx_v.at[g]], rows_v, gsem).wait()

            def scale_row(r, c2):
                for c in range(D_MODEL // L):
                    sl = pl.ds(c * L, L)
                    rows_v[r, sl] = rows_v[r, sl] * SCALE
                return c2

            lax.fori_loop(0, G, scale_row, 0)
            pltpu.sync_copy(rows_v, out_hbm.at[wid, g])
            return carry

        lax.fori_loop(0, n_groups, group_body, 0)

    return emb_kernel


def kernel(x, lut):
    b0, b1 = x.shape
    b = b0 * b1
    assert b % (NW * G) == 0
    n_groups = b // (NW * G)
    xr = x.reshape(NW, n_groups, G).astype(jnp.int32)
    out = _build(n_groups)(xr, lut)
    return out.reshape(b0, b1, D_MODEL)


# speedup vs baseline: 1.2091x; 1.2091x over previous
import functools

import jax
import jax.numpy as jnp
from jax import lax
from jax.experimental import pallas as pl
from jax.experimental.pallas import tpu as pltpu
from jax.experimental.pallas import tpu_sc as plsc

D_MODEL = 64
SCALE = 8.0
G = 128
KG = 5
NW = 32
L = 16
RC = KG * G


def _build(n_chunks):
    n_pairs = n_chunks // 2
    mesh = plsc.VectorSubcoreMesh(core_axis_name="c", subcore_axis_name="s")

    @functools.partial(
        pl.kernel,
        mesh=mesh,
        compiler_params=pltpu.CompilerParams(use_tc_tiling_on_sc=False),
        out_type=jax.ShapeDtypeStruct((NW, n_chunks, RC, D_MODEL), jnp.float32),
        scratch_types=[
            pltpu.VMEM((n_chunks, KG, G), jnp.int32),
            pltpu.VMEM((2, RC, D_MODEL), jnp.float32),
            pltpu.SemaphoreType.DMA,
            pltpu.SemaphoreType.DMA,
            pltpu.SemaphoreType.DMA,
        ],
    )
    def emb_kernel(x_hbm, lut_hbm, out_hbm, idx_v, bufs, gsem0, gsem1, osem):
        wid = lax.axis_index("s") * 2 + lax.axis_index("c")
        gsems = (gsem0, gsem1)
        pltpu.sync_copy(x_hbm.at[wid], idx_v)

        def fire_gathers(s, b):
            for j in range(KG):
                pltpu.async_copy(
                    lut_hbm.at[idx_v.at[s, j]],
                    bufs.at[b, pl.ds(j * G, G)],
                    gsems[b],
                )

        def drain_gathers(b):
            pltpu.make_async_copy(out_hbm.at[wid, 0], bufs.at[b], gsems[b]).wait()

        def fire_write(s, b):
            pltpu.async_copy(bufs.at[b], out_hbm.at[wid, s], osem)

        def drain_write():
            pltpu.make_async_copy(out_hbm.at[wid, 0], bufs.at[0], osem).wait()

        def scale(b):
            def body(i, c):
                r = i * 4
                for dr in range(4):
                    for cc in range(D_MODEL // L):
                        sl = pl.ds(cc * L, L)
                        bufs[b, r + dr, sl] = bufs[b, r + dr, sl] * SCALE
                return c

            lax.fori_loop(0, RC // 4, body, 0)

        fire_gathers(0, 0)

        def pair_body(t, carry):
            s0 = 2 * t

            @pl.when(t > 0)
            def _():
                drain_write()

            fire_gathers(s0 + 1, 1)
            drain_gathers(0)
            scale(0)
            fire_write(s0, 0)

            @pl.when(t < n_pairs - 1)
            def _():
                drain_write()
                fire_gathers(s0 + 2, 0)

            drain_gathers(1)
            scale(1)
            fire_write(s0 + 1, 1)
            return carry

        lax.fori_loop(0, n_pairs, pair_body, 0)
        drain_write()
        drain_write()

    return emb_kernel


def kernel(x, lut):
    b0, b1 = x.shape
    b = b0 * b1
    assert b % (NW * RC * 2) == 0
    n_chunks = b // (NW * RC)
    xr = x.reshape(NW, n_chunks, KG, G).astype(jnp.int32)
    out = _build(n_chunks)(xr, lut)
    return out.reshape(b0, b1, D_MODEL)
